# Initial kernel scaffold; baseline (speedup 1.0000x reference)
import jax
import jax.numpy as jnp
from jax.experimental import pallas as pl


def kernel(nv, ns, edge_index, atomic_number, Wv, Ws):
    raise NotImplementedError("write your pallas kernel here")


# speedup vs baseline: 163.0169x; 163.0169x over previous
import functools

import jax
import jax.numpy as jnp
from jax import lax
from jax.experimental import pallas as pl
from jax.experimental.pallas import tpu as pltpu
from jax.experimental.pallas import tpu_sc as plsc

N_NODES = 10000
N_EDGES = 160000
D = 128
N_TYPES = 4

NC = 2
NS = 16
EPT = N_EDGES // NS
B = 125
NB = EPT // B
ROWS_PT = 640
N_PAD = NS * ROWS_PT

_sc_mesh = plsc.VectorSubcoreMesh(core_axis_name="c", subcore_axis_name="s")


@functools.partial(
    pl.kernel,
    out_type=[jax.ShapeDtypeStruct((N_PAD, D), jnp.float32)] * 4,
    mesh=_sc_mesh,
    scratch_types=[
        pltpu.VMEM((NB, B), jnp.int32),
        pltpu.VMEM((NB, B), jnp.int32),
        pltpu.VMEM((B, D), jnp.float32),
        pltpu.VMEM_SHARED((N_PAD, D), jnp.float32),
        pltpu.SemaphoreType.DMA,
    ],
)
def _sc_segment_sum(x0, x1, x2, x3, src_hbm, dst_hbm, zeros_hbm,
                    o0, o1, o2, o3, src_v, dst_v, rows_v, acc, sem):
    c = lax.axis_index("c")
    s = lax.axis_index("s")

    pltpu.sync_copy(src_hbm.at[s], src_v)
    pltpu.sync_copy(dst_hbm.at[s], dst_v)

    def process_chunk(table, out):
        pltpu.sync_copy(zeros_hbm, acc.at[pl.ds(s * ROWS_PT, ROWS_PT)])
        plsc.subcore_barrier()
        for b in range(NB):
            pltpu.async_copy(table.at[src_v.at[b]], rows_v, sem).wait()
            pltpu.sync_copy(rows_v, acc.at[dst_v.at[b]], add=True)
        plsc.subcore_barrier()
        pltpu.sync_copy(acc.at[pl.ds(s * ROWS_PT, ROWS_PT)],
                        out.at[pl.ds(s * ROWS_PT, ROWS_PT)])

    @pl.when(c == 0)
    def _():
        process_chunk(x0, o0)
        process_chunk(x1, o1)

    @pl.when(c == 1)
    def _():
        process_chunk(x2, o2)
        process_chunk(x3, o3)


_R = 1000
_NBLK = N_NODES // _R


def _tc_body(a0, a1, a2, a3, tf, wv, ws, vout, sout):
    x = (a0[...], a1[...], a2[...], a3[...])
    t = tf[...]
    accv = [jnp.zeros((_R, D), jnp.float32) for _ in range(3)]
    accs = jnp.zeros((_R, D), jnp.float32)
    for ty in range(N_TYPES):
        m = jnp.where(t == float(ty), 1.0, 0.0)
        for k in range(3):
            accv[k] += m * jnp.dot(x[k], wv[ty],
                                   preferred_element_type=jnp.float32)
        accs += m * jnp.dot(x[3], ws[ty], preferred_element_type=jnp.float32)
    vout[...] = jnp.stack(accv, axis=1) * (1.0 / N_TYPES)
    sout[...] = (accs * jax.nn.sigmoid(accs)) * (1.0 / N_TYPES)


_row_spec = pl.BlockSpec((_R, D), lambda i: (i, 0))

_tc_call = pl.pallas_call(
    _tc_body,
    grid=(_NBLK,),
    in_specs=[_row_spec, _row_spec, _row_spec, _row_spec, _row_spec,
              pl.BlockSpec((N_TYPES, D, D), lambda i: (0, 0, 0)),
              pl.BlockSpec((N_TYPES, D, D), lambda i: (0, 0, 0))],
    out_specs=[pl.BlockSpec((_R, 3, D), lambda i: (i, 0, 0)),
               pl.BlockSpec((_R, D), lambda i: (i, 0))],
    out_shape=[jax.ShapeDtypeStruct((N_NODES, 3, D), jnp.float32),
               jax.ShapeDtypeStruct((N_NODES, D), jnp.float32)],
)


@jax.jit
def kernel(nv, ns, edge_index, atomic_number, Wv, Ws):
    src = edge_index[0].reshape(NS, NB, B)
    dst = edge_index[1].reshape(NS, NB, B)
    x0 = nv[:, 0, :]
    x1 = nv[:, 1, :]
    x2 = nv[:, 2, :]
    zeros = jnp.zeros((ROWS_PT, D), jnp.float32)
    a0, a1, a2, a3 = _sc_segment_sum(x0, x1, x2, ns, src, dst, zeros)
    tf = jnp.broadcast_to(atomic_number.astype(jnp.float32)[:, None],
                          (N_NODES, D))
    v_out, s_out = _tc_call(a0, a1, a2, a3, tf, Wv, Ws)
    return (v_out, s_out)
